# Initial kernel scaffold; baseline (speedup 1.0000x reference)
import jax
import jax.numpy as jnp
from jax.experimental import pallas as pl


def kernel(x, edge_index):
    raise NotImplementedError("write your pallas kernel here")


# speedup vs baseline: 8.5470x; 8.5470x over previous
import dataclasses
import functools

import jax
import jax.numpy as jnp
from jax import lax
from jax.experimental import pallas as pl
from jax.experimental.pallas import tpu as pltpu
from jax.experimental.pallas import tpu_sc as plsc

NC = 2
NS = 16
L = 16
B = 128


def _round_up(a: int, b: int) -> int:
    return (a + b - 1) // b * b


@functools.lru_cache(maxsize=None)
def _degree_call(n_pad: int, e_pad: int):
    mesh = plsc.VectorSubcoreMesh(core_axis_name="c", subcore_axis_name="s")
    e_sc = e_pad // NC
    e_tile = e_sc // NS
    nblk = e_tile // B

    cp = pltpu.CompilerParams()
    if "needs_layout_passes" in pltpu.CompilerParams.__dataclass_fields__:
        cp = dataclasses.replace(cp, needs_layout_passes=False)

    out_sds = jax.ShapeDtypeStruct((NC, NS, n_pad), jnp.int32)

    @functools.partial(
        pl.kernel,
        out_type=(out_sds, out_sds),
        mesh=mesh,
        compiler_params=cp,
        scratch_types=[
            pltpu.VMEM((B,), jnp.int32),
            pltpu.VMEM((B,), jnp.int32),
            pltpu.VMEM((n_pad,), jnp.int32),
            pltpu.VMEM((n_pad,), jnp.int32),
            pltpu.SemaphoreType.DMA,
        ],
    )
    def deg_kernel(srcp_hbm, dstp_hbm, out_s_hbm, out_d_hbm,
                   sidx_v, didx_v, hs_v, hd_v, sem):
        c = lax.axis_index("c")
        s = lax.axis_index("s")

        @pl.loop(jnp.int32(0), jnp.int32(n_pad // L))
        def _zero(i):
            hs_v[pl.ds(i * jnp.int32(L), L)] = jnp.zeros((L,), jnp.int32)
            hd_v[pl.ds(i * jnp.int32(L), L)] = jnp.zeros((L,), jnp.int32)

        base_e = c * jnp.int32(e_sc) + s * jnp.int32(e_tile)

        @pl.loop(jnp.int32(0), jnp.int32(nblk))
        def _edges(b):
            off = base_e + b * jnp.int32(B)
            pltpu.sync_copy(srcp_hbm.at[pl.ds(off, B)], sidx_v)
            pltpu.sync_copy(dstp_hbm.at[pl.ds(off, B)], didx_v)

            @pl.loop(jnp.int32(0), jnp.int32(B // L))
            def _vec(j):
                vs = sidx_v[pl.ds(j * jnp.int32(L), L)]
                cs, ms = plsc.scan_count(vs)
                plsc.addupdate_scatter(hs_v, [vs], cs, mask=ms)
                vd = didx_v[pl.ds(j * jnp.int32(L), L)]
                cd, md = plsc.scan_count(vd)
                plsc.addupdate_scatter(hd_v, [vd], cd, mask=md)

        pltpu.async_copy(hs_v, out_s_hbm.at[c, s], sem).wait()
        pltpu.async_copy(hd_v, out_d_hbm.at[c, s], sem).wait()

    return deg_kernel


@functools.lru_cache(maxsize=None)
def _agg_call(n_pad: int, e_pad: int, d: int):
    mesh = plsc.VectorSubcoreMesh(core_axis_name="c", subcore_axis_name="s")
    rows_pt = n_pad // NS
    e_sc = e_pad // NC
    e_tile = e_sc // NS
    nblk = e_tile // B

    @functools.partial(
        pl.kernel,
        out_type=jax.ShapeDtypeStruct((NC, n_pad, d), jnp.float32),
        mesh=mesh,
        scratch_types=[
            pltpu.VMEM((B, d), jnp.float32),
            pltpu.VMEM((B,), jnp.int32),
            pltpu.VMEM((B,), jnp.int32),
            pltpu.VMEM_SHARED((n_pad, d), jnp.float32),
            pltpu.SemaphoreType.DMA,
        ],
    )
    def agg_kernel(feat_hbm, srcp_hbm, dstp_hbm, out_hbm,
                   rows_v, sidx_v, didx_v, h_s, sem):
        c = lax.axis_index("c")
        s = lax.axis_index("s")

        @pl.loop(jnp.int32(0), jnp.int32(B))
        def _init(i):

            @pl.loop(jnp.int32(0), jnp.int32(d // L))
            def _initrow(j):
                rows_v[i, pl.ds(j * jnp.int32(L), L)] = jnp.zeros(
                    (L,), jnp.float32)

        @pl.loop(jnp.int32(0), jnp.int32(rows_pt // B))
        def _zero(k):
            base = s * jnp.int32(rows_pt) + k * jnp.int32(B)
            pltpu.sync_copy(rows_v, h_s.at[pl.ds(base, B), :])

        plsc.subcore_barrier()

        base_e = c * jnp.int32(e_sc) + s * jnp.int32(e_tile)

        @pl.loop(jnp.int32(0), jnp.int32(nblk))
        def _edges(b):
            off = base_e + b * jnp.int32(B)
            pltpu.sync_copy(srcp_hbm.at[pl.ds(off, B)], sidx_v)
            pltpu.sync_copy(dstp_hbm.at[pl.ds(off, B)], didx_v)
            pltpu.async_copy(feat_hbm.at[sidx_v], rows_v, sem).wait()
            pltpu.sync_copy(rows_v, h_s.at[didx_v], add=True)

        plsc.subcore_barrier()

        r0 = s * jnp.int32(rows_pt)
        pltpu.async_copy(h_s.at[pl.ds(r0, rows_pt), :],
                         out_hbm.at[c, pl.ds(r0, rows_pt), :], sem).wait()

    return agg_kernel


def kernel(x, edge_index):
    n, d = x.shape
    e = edge_index.shape[1]
    src = edge_index[0].astype(jnp.int32)
    dst = edge_index[1].astype(jnp.int32)

    n_pad = _round_up(n + 1, NS * B)
    e_pad = _round_up(e, NC * NS * B)
    pr = n_pad - n
    pad = e_pad - e
    pad_idx = n + (jnp.arange(pad, dtype=jnp.int32) % pr)
    srcp = jnp.concatenate([src, pad_idx])
    dstp = jnp.concatenate([dst, pad_idx])

    cnt_s, cnt_d = _degree_call(n_pad, e_pad)(srcp, dstp)
    deg_out = cnt_s.sum(axis=(0, 1))[:n].astype(jnp.float32) + 1.0
    deg_in = cnt_d.sum(axis=(0, 1))[:n].astype(jnp.float32) + 1.0

    feat = x * lax.rsqrt(deg_out)[:, None]
    featp = jnp.concatenate([feat, jnp.zeros((pr, d), jnp.float32)])

    hp = _agg_call(n_pad, e_pad, d)(featp, srcp, dstp)
    h = (hp[0, :n] + hp[1, :n] + feat) * lax.rsqrt(deg_in)[:, None]
    return h


# speedup vs baseline: 10.8816x; 1.2732x over previous
import dataclasses
import functools

import jax
import jax.numpy as jnp
from jax import lax
from jax.experimental import pallas as pl
from jax.experimental.pallas import tpu as pltpu
from jax.experimental.pallas import tpu_sc as plsc

NC = 2
NS = 16
L = 16
B = 128


def _round_up(a: int, b: int) -> int:
    return (a + b - 1) // b * b


@functools.lru_cache(maxsize=None)
def _degree_call(n_pad: int, e_pad: int):
    mesh = plsc.VectorSubcoreMesh(core_axis_name="c", subcore_axis_name="s")
    e_sc = e_pad // NC
    e_tile = e_sc // NS
    nblk = e_tile // B

    cp = pltpu.CompilerParams()
    if "needs_layout_passes" in pltpu.CompilerParams.__dataclass_fields__:
        cp = dataclasses.replace(cp, needs_layout_passes=False)

    out_sds = jax.ShapeDtypeStruct((NC, NS, n_pad), jnp.int32)

    @functools.partial(
        pl.kernel,
        out_type=(out_sds, out_sds),
        mesh=mesh,
        compiler_params=cp,
        scratch_types=[
            pltpu.VMEM((B,), jnp.int32),
            pltpu.VMEM((B,), jnp.int32),
            pltpu.VMEM((n_pad,), jnp.int32),
            pltpu.VMEM((n_pad,), jnp.int32),
            pltpu.SemaphoreType.DMA,
        ],
    )
    def deg_kernel(srcp_hbm, dstp_hbm, out_s_hbm, out_d_hbm,
                   sidx_v, didx_v, hs_v, hd_v, sem):
        c = lax.axis_index("c")
        s = lax.axis_index("s")

        @pl.loop(jnp.int32(0), jnp.int32(n_pad // L))
        def _zero(i):
            hs_v[pl.ds(i * jnp.int32(L), L)] = jnp.zeros((L,), jnp.int32)
            hd_v[pl.ds(i * jnp.int32(L), L)] = jnp.zeros((L,), jnp.int32)

        base_e = c * jnp.int32(e_sc) + s * jnp.int32(e_tile)

        @pl.loop(jnp.int32(0), jnp.int32(nblk))
        def _edges(b):
            off = base_e + b * jnp.int32(B)
            pltpu.sync_copy(srcp_hbm.at[pl.ds(off, B)], sidx_v)
            pltpu.sync_copy(dstp_hbm.at[pl.ds(off, B)], didx_v)

            @pl.loop(jnp.int32(0), jnp.int32(B // L))
            def _vec(j):
                vs = sidx_v[pl.ds(j * jnp.int32(L), L)]
                cs, ms = plsc.scan_count(vs)
                plsc.addupdate_scatter(hs_v, [vs], cs, mask=ms)
                vd = didx_v[pl.ds(j * jnp.int32(L), L)]
                cd, md = plsc.scan_count(vd)
                plsc.addupdate_scatter(hd_v, [vd], cd, mask=md)

        pltpu.async_copy(hs_v, out_s_hbm.at[c, s], sem).wait()
        pltpu.async_copy(hd_v, out_d_hbm.at[c, s], sem).wait()

    return deg_kernel


@functools.lru_cache(maxsize=None)
def _agg_call(n_pad: int, e_pad: int, d: int):
    mesh = plsc.VectorSubcoreMesh(core_axis_name="c", subcore_axis_name="s")
    rows_pt = n_pad // NS
    e_sc = e_pad // NC
    e_tile = e_sc // NS
    nblk = e_tile // B
    assert nblk % 2 == 0

    @functools.partial(
        pl.kernel,
        out_type=jax.ShapeDtypeStruct((NC, n_pad, d), jnp.float32),
        mesh=mesh,
        scratch_types=[
            pltpu.VMEM((B, d), jnp.float32),
            pltpu.VMEM((B, d), jnp.float32),
            pltpu.VMEM((B,), jnp.int32),
            pltpu.VMEM((B,), jnp.int32),
            pltpu.VMEM((B,), jnp.int32),
            pltpu.VMEM((B,), jnp.int32),
            pltpu.VMEM_SHARED((n_pad, d), jnp.float32),
            pltpu.SemaphoreType.DMA,
            pltpu.SemaphoreType.DMA,
        ],
    )
    def agg_kernel(feat_hbm, srcp_hbm, dstp_hbm, out_hbm,
                   rows0_v, rows1_v, sidx0_v, sidx1_v, didx0_v, didx1_v,
                   h_s, sem0, sem1):
        c = lax.axis_index("c")
        s = lax.axis_index("s")
        rows = (rows0_v, rows1_v)
        sidx = (sidx0_v, sidx1_v)
        didx = (didx0_v, didx1_v)
        sems = (sem0, sem1)

        @pl.loop(jnp.int32(0), jnp.int32(B))
        def _init(i):
            @pl.loop(jnp.int32(0), jnp.int32(d // L))
            def _initrow(j):
                rows0_v[i, pl.ds(j * jnp.int32(L), L)] = jnp.zeros(
                    (L,), jnp.float32)

        @pl.loop(jnp.int32(0), jnp.int32(rows_pt // B))
        def _zero(k):
            base = s * jnp.int32(rows_pt) + k * jnp.int32(B)
            pltpu.sync_copy(rows0_v, h_s.at[pl.ds(base, B), :])

        plsc.subcore_barrier()

        base_e = c * jnp.int32(e_sc) + s * jnp.int32(e_tile)

        def load_and_gather(q, blk):
            off = base_e + blk * jnp.int32(B)
            pltpu.sync_copy(srcp_hbm.at[pl.ds(off, B)], sidx[q])
            pltpu.sync_copy(dstp_hbm.at[pl.ds(off, B)], didx[q])
            return pltpu.async_copy(feat_hbm.at[sidx[q]], rows[q], sems[q])

        load_and_gather(0, jnp.int32(0))
        load_and_gather(1, jnp.int32(1))

        @pl.loop(jnp.int32(0), jnp.int32(nblk // 2))
        def _edges(p):
            for q in range(2):
                pltpu.make_async_copy(feat_hbm.at[sidx[q]], rows[q],
                                      sems[q]).wait()
                pltpu.sync_copy(rows[q], h_s.at[didx[q]], add=True)
                blk = jnp.int32(2) * p + jnp.int32(q + 2)
                load_and_gather(q, blk)

        pltpu.make_async_copy(feat_hbm.at[sidx[0]], rows[0], sems[0]).wait()
        pltpu.make_async_copy(feat_hbm.at[sidx[1]], rows[1], sems[1]).wait()

        plsc.subcore_barrier()

        r0 = s * jnp.int32(rows_pt)
        pltpu.async_copy(h_s.at[pl.ds(r0, rows_pt), :],
                         out_hbm.at[c, pl.ds(r0, rows_pt), :], sem0).wait()

    return agg_kernel


def kernel(x, edge_index):
    n, d = x.shape
    e = edge_index.shape[1]
    src = edge_index[0].astype(jnp.int32)
    dst = edge_index[1].astype(jnp.int32)

    n_pad = _round_up(n + 1, NS * B)
    e_pad = _round_up(e, 2 * NC * NS * B)
    pr = n_pad - n
    pad = e_pad - e
    pad_idx = n + (jnp.arange(pad, dtype=jnp.int32) % pr)
    extra = jnp.zeros((2 * B,), jnp.int32)
    srcp = jnp.concatenate([src, pad_idx, extra])
    dstp = jnp.concatenate([dst, pad_idx, extra])

    cnt_s, cnt_d = _degree_call(n_pad, e_pad)(srcp, dstp)
    deg_out = cnt_s.sum(axis=(0, 1))[:n].astype(jnp.float32) + 1.0
    deg_in = cnt_d.sum(axis=(0, 1))[:n].astype(jnp.float32) + 1.0

    feat = x * lax.rsqrt(deg_out)[:, None]
    featp = jnp.concatenate([feat, jnp.zeros((pr, d), jnp.float32)])

    hp = _agg_call(n_pad, e_pad, d)(featp, srcp, dstp)
    h = (hp[0, :n] + hp[1, :n] + feat) * lax.rsqrt(deg_in)[:, None]
    return h


# speedup vs baseline: 14.1987x; 1.3048x over previous
import dataclasses
import functools

import jax
import jax.numpy as jnp
from jax import lax
from jax.experimental import pallas as pl
from jax.experimental.pallas import tpu as pltpu
from jax.experimental.pallas import tpu_sc as plsc

NC = 2
NS = 16
L = 16
B = 128


def _round_up(a: int, b: int) -> int:
    return (a + b - 1) // b * b


@functools.lru_cache(maxsize=None)
def _degree_call(n_pad: int, e_pad: int):
    mesh = plsc.VectorSubcoreMesh(core_axis_name="c", subcore_axis_name="s")
    SB = 1024
    e_sc = e_pad // NC
    e_tile = e_sc // NS
    nsb = e_tile // SB
    assert nsb % 2 == 0

    cp = pltpu.CompilerParams()
    if "needs_layout_passes" in pltpu.CompilerParams.__dataclass_fields__:
        cp = dataclasses.replace(cp, needs_layout_passes=False)

    out_sds = jax.ShapeDtypeStruct((NC, NS, n_pad), jnp.int32)

    @functools.partial(
        pl.kernel,
        out_type=(out_sds, out_sds),
        mesh=mesh,
        compiler_params=cp,
        scratch_types=[
            pltpu.VMEM((SB,), jnp.int32),
            pltpu.VMEM((SB,), jnp.int32),
            pltpu.VMEM((SB,), jnp.int32),
            pltpu.VMEM((SB,), jnp.int32),
            pltpu.VMEM((n_pad,), jnp.int32),
            pltpu.VMEM((n_pad,), jnp.int32),
            pltpu.SemaphoreType.DMA,
            pltpu.SemaphoreType.DMA,
            pltpu.SemaphoreType.DMA,
            pltpu.SemaphoreType.DMA,
        ],
    )
    def deg_kernel(srcp_hbm, dstp_hbm, out_s_hbm, out_d_hbm,
                   sidx0_v, sidx1_v, didx0_v, didx1_v, hs_v, hd_v,
                   sem_s0, sem_s1, sem_d0, sem_d1):
        c = lax.axis_index("c")
        s = lax.axis_index("s")
        sidx = (sidx0_v, sidx1_v)
        didx = (didx0_v, didx1_v)
        sem_s = (sem_s0, sem_s1)
        sem_d = (sem_d0, sem_d1)

        @pl.loop(jnp.int32(0), jnp.int32(n_pad // L))
        def _zero(i):
            hs_v[pl.ds(i * jnp.int32(L), L)] = jnp.zeros((L,), jnp.int32)
            hd_v[pl.ds(i * jnp.int32(L), L)] = jnp.zeros((L,), jnp.int32)

        base_e = c * jnp.int32(e_sc) + s * jnp.int32(e_tile)

        def start_loads(q, sb):
            off = base_e + sb * jnp.int32(SB)
            pltpu.async_copy(srcp_hbm.at[pl.ds(off, SB)], sidx[q], sem_s[q])
            pltpu.async_copy(dstp_hbm.at[pl.ds(off, SB)], didx[q], sem_d[q])

        def wait_loads(q):
            pltpu.make_async_copy(srcp_hbm.at[pl.ds(0, SB)], sidx[q],
                                  sem_s[q]).wait()
            pltpu.make_async_copy(dstp_hbm.at[pl.ds(0, SB)], didx[q],
                                  sem_d[q]).wait()

        start_loads(0, jnp.int32(0))
        start_loads(1, jnp.int32(1))

        @pl.loop(jnp.int32(0), jnp.int32(nsb // 2))
        def _super(p):
            for q in range(2):
                wait_loads(q)

                @pl.loop(jnp.int32(0), jnp.int32(SB // L))
                def _vec(j):
                    vs = sidx[q][pl.ds(j * jnp.int32(L), L)]
                    cs, ms = plsc.scan_count(vs)
                    plsc.addupdate_scatter(hs_v, [vs], cs, mask=ms)
                    vd = didx[q][pl.ds(j * jnp.int32(L), L)]
                    cd, md = plsc.scan_count(vd)
                    plsc.addupdate_scatter(hd_v, [vd], cd, mask=md)

                sb = jnp.int32(2) * p + jnp.int32(q + 2)
                start_loads(q, sb)

        wait_loads(0)
        wait_loads(1)

        pltpu.async_copy(hs_v, out_s_hbm.at[c, s], sem_s0).wait()
        pltpu.async_copy(hd_v, out_d_hbm.at[c, s], sem_d0).wait()

    return deg_kernel


@functools.lru_cache(maxsize=None)
def _agg_call(n_pad: int, e_pad: int, d: int):
    mesh = plsc.VectorSubcoreMesh(core_axis_name="c", subcore_axis_name="s")
    rows_pt = n_pad // NS
    e_sc = e_pad // NC
    e_tile = e_sc // NS
    nblk = e_tile // B
    assert nblk % 2 == 0

    @functools.partial(
        pl.kernel,
        out_type=jax.ShapeDtypeStruct((NC, n_pad, d), jnp.float32),
        mesh=mesh,
        scratch_types=[
            pltpu.VMEM((B, d), jnp.float32),
            pltpu.VMEM((B, d), jnp.float32),
            pltpu.VMEM((B,), jnp.int32),
            pltpu.VMEM((B,), jnp.int32),
            pltpu.VMEM((B,), jnp.int32),
            pltpu.VMEM((B,), jnp.int32),
            pltpu.VMEM_SHARED((n_pad, d), jnp.float32),
            pltpu.SemaphoreType.DMA,
            pltpu.SemaphoreType.DMA,
        ],
    )
    def agg_kernel(feat_hbm, srcp_hbm, dstp_hbm, out_hbm,
                   rows0_v, rows1_v, sidx0_v, sidx1_v, didx0_v, didx1_v,
                   h_s, sem0, sem1):
        c = lax.axis_index("c")
        s = lax.axis_index("s")
        rows = (rows0_v, rows1_v)
        sidx = (sidx0_v, sidx1_v)
        didx = (didx0_v, didx1_v)
        sems = (sem0, sem1)

        @pl.loop(jnp.int32(0), jnp.int32(B))
        def _init(i):
            @pl.loop(jnp.int32(0), jnp.int32(d // L))
            def _initrow(j):
                rows0_v[i, pl.ds(j * jnp.int32(L), L)] = jnp.zeros(
                    (L,), jnp.float32)

        @pl.loop(jnp.int32(0), jnp.int32(rows_pt // B))
        def _zero(k):
            base = s * jnp.int32(rows_pt) + k * jnp.int32(B)
            pltpu.sync_copy(rows0_v, h_s.at[pl.ds(base, B), :])

        plsc.subcore_barrier()

        base_e = c * jnp.int32(e_sc) + s * jnp.int32(e_tile)

        def load_and_gather(q, blk):
            off = base_e + blk * jnp.int32(B)
            pltpu.sync_copy(srcp_hbm.at[pl.ds(off, B)], sidx[q])
            pltpu.sync_copy(dstp_hbm.at[pl.ds(off, B)], didx[q])
            return pltpu.async_copy(feat_hbm.at[sidx[q]], rows[q], sems[q])

        load_and_gather(0, jnp.int32(0))
        load_and_gather(1, jnp.int32(1))

        @pl.loop(jnp.int32(0), jnp.int32(nblk // 2))
        def _edges(p):
            for q in range(2):
                pltpu.make_async_copy(feat_hbm.at[sidx[q]], rows[q],
                                      sems[q]).wait()
                pltpu.sync_copy(rows[q], h_s.at[didx[q]], add=True)
                blk = jnp.int32(2) * p + jnp.int32(q + 2)
                load_and_gather(q, blk)

        pltpu.make_async_copy(feat_hbm.at[sidx[0]], rows[0], sems[0]).wait()
        pltpu.make_async_copy(feat_hbm.at[sidx[1]], rows[1], sems[1]).wait()

        plsc.subcore_barrier()

        r0 = s * jnp.int32(rows_pt)
        pltpu.async_copy(h_s.at[pl.ds(r0, rows_pt), :],
                         out_hbm.at[c, pl.ds(r0, rows_pt), :], sem0).wait()

    return agg_kernel


def kernel(x, edge_index):
    n, d = x.shape
    e = edge_index.shape[1]
    src = edge_index[0].astype(jnp.int32)
    dst = edge_index[1].astype(jnp.int32)

    n_pad = _round_up(n + 1, NS * B)
    e_pad = _round_up(e, 2 * NC * NS * B)
    pr = n_pad - n
    pad = e_pad - e
    pad_idx = n + (jnp.arange(pad, dtype=jnp.int32) % pr)
    extra = jnp.zeros((2048,), jnp.int32)
    srcp = jnp.concatenate([src, pad_idx, extra])
    dstp = jnp.concatenate([dst, pad_idx, extra])

    cnt_s, cnt_d = _degree_call(n_pad, e_pad)(srcp, dstp)
    deg_out = cnt_s.sum(axis=(0, 1))[:n].astype(jnp.float32) + 1.0
    deg_in = cnt_d.sum(axis=(0, 1))[:n].astype(jnp.float32) + 1.0

    feat = x * lax.rsqrt(deg_out)[:, None]
    featp = jnp.concatenate([feat, jnp.zeros((pr, d), jnp.float32)])

    hp = _agg_call(n_pad, e_pad, d)(featp, srcp, dstp)
    h = (hp[0, :n] + hp[1, :n] + feat) * lax.rsqrt(deg_in)[:, None]
    return h


# speedup vs baseline: 18.8901x; 1.3304x over previous
import dataclasses
import functools

import jax
import jax.numpy as jnp
from jax import lax
from jax.experimental import pallas as pl
from jax.experimental.pallas import tpu as pltpu
from jax.experimental.pallas import tpu_sc as plsc

NC = 2
NS = 16
L = 16
B = 128


def _round_up(a: int, b: int) -> int:
    return (a + b - 1) // b * b


@functools.lru_cache(maxsize=None)
def _degree_call(n_pad: int, e_pad: int):
    mesh = plsc.VectorSubcoreMesh(core_axis_name="c", subcore_axis_name="s")
    SB = 1024
    e_sc = e_pad // NC
    e_tile = e_sc // NS
    nsb = e_tile // SB
    assert nsb % 2 == 0

    cp = pltpu.CompilerParams()
    if "needs_layout_passes" in pltpu.CompilerParams.__dataclass_fields__:
        cp = dataclasses.replace(cp, needs_layout_passes=False)

    out_sds = jax.ShapeDtypeStruct((NC, NS, n_pad), jnp.int32)

    @functools.partial(
        pl.kernel,
        out_type=(out_sds, out_sds),
        mesh=mesh,
        compiler_params=cp,
        scratch_types=[
            pltpu.VMEM((SB,), jnp.int32),
            pltpu.VMEM((SB,), jnp.int32),
            pltpu.VMEM((SB,), jnp.int32),
            pltpu.VMEM((SB,), jnp.int32),
            pltpu.VMEM((n_pad,), jnp.int32),
            pltpu.VMEM((n_pad,), jnp.int32),
            pltpu.SemaphoreType.DMA,
            pltpu.SemaphoreType.DMA,
            pltpu.SemaphoreType.DMA,
            pltpu.SemaphoreType.DMA,
        ],
    )
    def deg_kernel(srcp_hbm, dstp_hbm, out_s_hbm, out_d_hbm,
                   sidx0_v, sidx1_v, didx0_v, didx1_v, hs_v, hd_v,
                   sem_s0, sem_s1, sem_d0, sem_d1):
        c = lax.axis_index("c")
        s = lax.axis_index("s")
        sidx = (sidx0_v, sidx1_v)
        didx = (didx0_v, didx1_v)
        sem_s = (sem_s0, sem_s1)
        sem_d = (sem_d0, sem_d1)

        @pl.loop(jnp.int32(0), jnp.int32(n_pad // (8 * L)))
        def _zero(i):
            base = i * jnp.int32(8 * L)
            for u in range(8):
                off = base + jnp.int32(u * L)
                hs_v[pl.ds(off, L)] = jnp.zeros((L,), jnp.int32)
                hd_v[pl.ds(off, L)] = jnp.zeros((L,), jnp.int32)

        base_e = c * jnp.int32(e_sc) + s * jnp.int32(e_tile)

        def start_loads(q, sb):
            off = base_e + sb * jnp.int32(SB)
            pltpu.async_copy(srcp_hbm.at[pl.ds(off, SB)], sidx[q], sem_s[q])
            pltpu.async_copy(dstp_hbm.at[pl.ds(off, SB)], didx[q], sem_d[q])

        def wait_loads(q):
            pltpu.make_async_copy(srcp_hbm.at[pl.ds(0, SB)], sidx[q],
                                  sem_s[q]).wait()
            pltpu.make_async_copy(dstp_hbm.at[pl.ds(0, SB)], didx[q],
                                  sem_d[q]).wait()

        start_loads(0, jnp.int32(0))
        start_loads(1, jnp.int32(1))

        @pl.loop(jnp.int32(0), jnp.int32(nsb // 2))
        def _super(p):
            for q in range(2):
                wait_loads(q)

                @pl.loop(jnp.int32(0), jnp.int32(SB // (8 * L)))
                def _vec(j):
                    ones = jnp.full((L,), 1, jnp.int32)
                    jbase = j * jnp.int32(8 * L)
                    for u in range(8):
                        off = jbase + jnp.int32(u * L)
                        vs = sidx[q][pl.ds(off, L)]
                        plsc.addupdate_scatter(hs_v, [vs], ones)
                        vd = didx[q][pl.ds(off, L)]
                        plsc.addupdate_scatter(hd_v, [vd], ones)

                sb = jnp.int32(2) * p + jnp.int32(q + 2)
                start_loads(q, sb)

        wait_loads(0)
        wait_loads(1)

        pltpu.async_copy(hs_v, out_s_hbm.at[c, s], sem_s0).wait()
        pltpu.async_copy(hd_v, out_d_hbm.at[c, s], sem_d0).wait()

    return deg_kernel


@functools.lru_cache(maxsize=None)
def _agg_call(n_pad: int, e_pad: int, d: int):
    mesh = plsc.VectorSubcoreMesh(core_axis_name="c", subcore_axis_name="s")
    rows_pt = n_pad // NS
    e_sc = e_pad // NC
    e_tile = e_sc // NS
    SBB = 8
    nsb = e_tile // (SBB * B)
    assert nsb % 2 == 0

    idx_t = pltpu.VMEM((SBB, B), jnp.int32)

    @functools.partial(
        pl.kernel,
        out_type=jax.ShapeDtypeStruct((NC, n_pad, d), jnp.float32),
        mesh=mesh,
        scratch_types=[
            pltpu.VMEM((B, d), jnp.float32),
            pltpu.VMEM((B, d), jnp.float32),
            idx_t, idx_t,
            idx_t, idx_t,
            pltpu.VMEM_SHARED((n_pad, d), jnp.float32),
            pltpu.SemaphoreType.DMA, pltpu.SemaphoreType.DMA,
            pltpu.SemaphoreType.DMA, pltpu.SemaphoreType.DMA,
            pltpu.SemaphoreType.DMA, pltpu.SemaphoreType.DMA,
        ],
    )
    def agg_kernel(feat_hbm, srcp_hbm, dstp_hbm, out_hbm,
                   rows0_v, rows1_v,
                   sidxa_v, sidxb_v, didxa_v, didxb_v, h_s,
                   gsem0, gsem1,
                   isem_sa, isem_sb, isem_da, isem_db):
        c = lax.axis_index("c")
        s = lax.axis_index("s")
        rows = (rows0_v, rows1_v)
        gsem = (gsem0, gsem1)
        sidx = (sidxa_v, sidxb_v)
        didx = (didxa_v, didxb_v)
        isem_s = (isem_sa, isem_sb)
        isem_d = (isem_da, isem_db)

        @pl.loop(jnp.int32(0), jnp.int32(B))
        def _init(i):
            for u in range(d // L):
                rows0_v[i, pl.ds(jnp.int32(u * L), L)] = jnp.zeros(
                    (L,), jnp.float32)

        @pl.loop(jnp.int32(0), jnp.int32(rows_pt // B))
        def _zero(k):
            base = s * jnp.int32(rows_pt) + k * jnp.int32(B)

            @pl.when(c == 0)
            def _seed():
                pltpu.sync_copy(feat_hbm.at[pl.ds(base, B), :],
                                h_s.at[pl.ds(base, B), :])

            @pl.when(c != 0)
            def _zero_fill():
                pltpu.sync_copy(rows0_v, h_s.at[pl.ds(base, B), :])

        plsc.subcore_barrier()

        base_row = (c * jnp.int32(e_sc) + s * jnp.int32(e_tile)) // jnp.int32(B)

        def start_idx(q, sb):
            r0 = pl.multiple_of(base_row + sb * jnp.int32(SBB), SBB)
            pltpu.async_copy(srcp_hbm.at[pl.ds(r0, SBB), :], sidx[q],
                             isem_s[q])
            pltpu.async_copy(dstp_hbm.at[pl.ds(r0, SBB), :], didx[q],
                             isem_d[q])

        def wait_idx(q):
            pltpu.make_async_copy(srcp_hbm.at[pl.ds(0, SBB), :], sidx[q],
                                  isem_s[q]).wait()
            pltpu.make_async_copy(dstp_hbm.at[pl.ds(0, SBB), :], didx[q],
                                  isem_d[q]).wait()

        def start_gather(slot, q, j):
            pltpu.async_copy(feat_hbm.at[sidx[q].at[jnp.int32(j)]],
                             rows[slot], gsem[slot])

        def wait_gather(slot, q, j):
            pltpu.make_async_copy(feat_hbm.at[sidx[q].at[jnp.int32(j)]],
                                  rows[slot], gsem[slot]).wait()

        start_idx(0, jnp.int32(0))
        start_idx(1, jnp.int32(1))
        wait_idx(0)
        for j in range(2):
            start_gather(j, 0, j)

        @pl.loop(jnp.int32(0), jnp.int32(nsb // 2))
        def _super(p):
            for q in range(2):
                sb = jnp.int32(2) * p + jnp.int32(q)
                for j in range(SBB):
                    slot = j % 2
                    wait_gather(slot, q, j)
                    pltpu.sync_copy(rows[slot],
                                    h_s.at[didx[q].at[jnp.int32(j)]],
                                    add=True)
                    if j < SBB - 2:
                        start_gather(slot, q, j + 2)
                    else:
                        if j == SBB - 2:
                            wait_idx(1 - q)
                        start_gather(slot, 1 - q, j - (SBB - 2))
                start_idx(q, sb + jnp.int32(2))

        for j in range(2):
            wait_gather(j, nsb % 2, j)
        wait_idx((nsb - 1) % 2)

        plsc.subcore_barrier()

        r0 = s * jnp.int32(rows_pt)
        pltpu.async_copy(h_s.at[pl.ds(r0, rows_pt), :],
                         out_hbm.at[c, pl.ds(r0, rows_pt), :], gsem0).wait()

    return agg_kernel


def kernel(x, edge_index):
    n, d = x.shape
    e = edge_index.shape[1]
    src = edge_index[0].astype(jnp.int32)
    dst = edge_index[1].astype(jnp.int32)

    n_pad = _round_up(n + 1, NS * B)
    e_pad = _round_up(e, 2 * NC * NS * B)
    pr = n_pad - n
    pad = e_pad - e
    pad_idx = n + (jnp.arange(pad, dtype=jnp.int32) % pr)
    extra = jnp.zeros((2048,), jnp.int32)
    srcp = jnp.concatenate([src, pad_idx, extra])
    dstp = jnp.concatenate([dst, pad_idx, extra])

    cnt_s, cnt_d = _degree_call(n_pad, e_pad)(srcp, dstp)
    deg_out = cnt_s.sum(axis=(0, 1))[:n].astype(jnp.float32) + 1.0
    deg_in = cnt_d.sum(axis=(0, 1))[:n].astype(jnp.float32) + 1.0

    feat = x * lax.rsqrt(deg_out)[:, None]
    featp = jnp.concatenate([feat, jnp.zeros((pr, d), jnp.float32)])

    srcp2 = srcp.reshape(-1, B)
    dstp2 = dstp.reshape(-1, B)
    hp = _agg_call(n_pad, e_pad, d)(featp, srcp2, dstp2)
    h = (hp[0, :n] + hp[1, :n]) * lax.rsqrt(deg_in)[:, None]
    return h


# speedup vs baseline: 18.9657x; 1.0040x over previous
import dataclasses
import functools

import jax
import jax.numpy as jnp
from jax import lax
from jax.experimental import pallas as pl
from jax.experimental.pallas import tpu as pltpu
from jax.experimental.pallas import tpu_sc as plsc

NC = 2
NS = 16
L = 16
B = 128


def _round_up(a: int, b: int) -> int:
    return (a + b - 1) // b * b


@functools.lru_cache(maxsize=None)
def _degree_call(n_pad: int, e_pad: int):
    mesh = plsc.VectorSubcoreMesh(core_axis_name="c", subcore_axis_name="s")
    SB = 1024
    e_sc = e_pad // NC
    e_tile = e_sc // NS
    nsb = e_tile // SB
    assert nsb % 2 == 0

    cp = pltpu.CompilerParams()
    if "needs_layout_passes" in pltpu.CompilerParams.__dataclass_fields__:
        cp = dataclasses.replace(cp, needs_layout_passes=False)

    @functools.partial(
        pl.kernel,
        out_type=jax.ShapeDtypeStruct((NC, NS, n_pad), jnp.int32),
        mesh=mesh,
        compiler_params=cp,
        scratch_types=[
            pltpu.VMEM((SB,), jnp.int32),
            pltpu.VMEM((SB,), jnp.int32),
            pltpu.VMEM((n_pad,), jnp.int32),
            pltpu.SemaphoreType.DMA,
            pltpu.SemaphoreType.DMA,
        ],
    )
    def deg_kernel(srcp_hbm, out_s_hbm, sidx0_v, sidx1_v, hs_v,
                   sem_s0, sem_s1):
        c = lax.axis_index("c")
        s = lax.axis_index("s")
        sidx = (sidx0_v, sidx1_v)
        sem_s = (sem_s0, sem_s1)

        @pl.loop(jnp.int32(0), jnp.int32(n_pad // (8 * L)))
        def _zero(i):
            base = i * jnp.int32(8 * L)
            for u in range(8):
                off = base + jnp.int32(u * L)
                hs_v[pl.ds(off, L)] = jnp.zeros((L,), jnp.int32)

        base_e = c * jnp.int32(e_sc) + s * jnp.int32(e_tile)

        def start_load(q, sb):
            off = base_e + sb * jnp.int32(SB)
            pltpu.async_copy(srcp_hbm.at[pl.ds(off, SB)], sidx[q], sem_s[q])

        def wait_load(q):
            pltpu.make_async_copy(srcp_hbm.at[pl.ds(0, SB)], sidx[q],
                                  sem_s[q]).wait()

        start_load(0, jnp.int32(0))
        start_load(1, jnp.int32(1))

        @pl.loop(jnp.int32(0), jnp.int32(nsb // 2))
        def _super(p):
            for q in range(2):
                wait_load(q)

                @pl.loop(jnp.int32(0), jnp.int32(SB // (8 * L)))
                def _vec(j):
                    ones = jnp.full((L,), 1, jnp.int32)
                    jbase = j * jnp.int32(8 * L)
                    for u in range(8):
                        off = jbase + jnp.int32(u * L)
                        vs = sidx[q][pl.ds(off, L)]
                        plsc.addupdate_scatter(hs_v, [vs], ones)

                sb = jnp.int32(2) * p + jnp.int32(q + 2)
                start_load(q, sb)

        wait_load(0)
        wait_load(1)

        pltpu.async_copy(hs_v, out_s_hbm.at[c, s], sem_s0).wait()

    return deg_kernel


@functools.lru_cache(maxsize=None)
def _agg_call(n_pad: int, e_pad: int, d: int):
    mesh = plsc.VectorSubcoreMesh(core_axis_name="c", subcore_axis_name="s")
    rows_pt = n_pad // NS
    e_sc = e_pad // NC
    e_tile = e_sc // NS
    SBB = 8
    nsb = e_tile // (SBB * B)
    assert nsb % 2 == 0

    idx_t = pltpu.VMEM((SBB, B), jnp.int32)

    cp = pltpu.CompilerParams()
    if "needs_layout_passes" in pltpu.CompilerParams.__dataclass_fields__:
        cp = dataclasses.replace(cp, needs_layout_passes=False)

    @functools.partial(
        pl.kernel,
        out_type=(jax.ShapeDtypeStruct((NC, n_pad, d), jnp.float32),
                  jax.ShapeDtypeStruct((NC, NS, n_pad), jnp.int32)),
        mesh=mesh,
        compiler_params=cp,
        scratch_types=[
            pltpu.VMEM((B, d), jnp.float32),
            pltpu.VMEM((B, d), jnp.float32),
            idx_t, idx_t,
            idx_t, idx_t,
            pltpu.VMEM((n_pad,), jnp.int32),
            pltpu.VMEM_SHARED((n_pad, d), jnp.float32),
            pltpu.SemaphoreType.DMA, pltpu.SemaphoreType.DMA,
            pltpu.SemaphoreType.DMA, pltpu.SemaphoreType.DMA,
            pltpu.SemaphoreType.DMA, pltpu.SemaphoreType.DMA,
            pltpu.SemaphoreType.DMA, pltpu.SemaphoreType.DMA,
        ],
    )
    def agg_kernel(feat_hbm, srcp_hbm, dstp_hbm, out_hbm, out_d_hbm,
                   rows0_v, rows1_v,
                   sidxa_v, sidxb_v, didxa_v, didxb_v, hd_v, h_s,
                   gsem0, gsem1,
                   isem_sa, isem_sb, isem_da, isem_db,
                   ssem0, ssem1):
        c = lax.axis_index("c")
        s = lax.axis_index("s")
        rows = (rows0_v, rows1_v)
        gsem = (gsem0, gsem1)
        ssem = (ssem0, ssem1)
        sidx = (sidxa_v, sidxb_v)
        didx = (didxa_v, didxb_v)
        isem_s = (isem_sa, isem_sb)
        isem_d = (isem_da, isem_db)

        @pl.loop(jnp.int32(0), jnp.int32(B))
        def _init(i):
            for u in range(d // L):
                rows0_v[i, pl.ds(jnp.int32(u * L), L)] = jnp.zeros(
                    (L,), jnp.float32)

        @pl.loop(jnp.int32(0), jnp.int32(n_pad // (8 * L)))
        def _zeroh(i):
            base = i * jnp.int32(8 * L)
            for u in range(8):
                hd_v[pl.ds(base + jnp.int32(u * L), L)] = jnp.zeros(
                    (L,), jnp.int32)

        @pl.loop(jnp.int32(0), jnp.int32(rows_pt // B))
        def _zero(k):
            base = s * jnp.int32(rows_pt) + k * jnp.int32(B)

            @pl.when(c == 0)
            def _seed():
                pltpu.sync_copy(feat_hbm.at[pl.ds(base, B), :],
                                h_s.at[pl.ds(base, B), :])

            @pl.when(c != 0)
            def _zero_fill():
                pltpu.sync_copy(rows0_v, h_s.at[pl.ds(base, B), :])

        plsc.subcore_barrier()

        base_row = (c * jnp.int32(e_sc) + s * jnp.int32(e_tile)) // jnp.int32(B)

        def start_idx(q, sb):
            r0 = pl.multiple_of(base_row + sb * jnp.int32(SBB), SBB)
            pltpu.async_copy(srcp_hbm.at[pl.ds(r0, SBB), :], sidx[q],
                             isem_s[q])
            pltpu.async_copy(dstp_hbm.at[pl.ds(r0, SBB), :], didx[q],
                             isem_d[q])

        def wait_idx(q):
            pltpu.make_async_copy(srcp_hbm.at[pl.ds(0, SBB), :], sidx[q],
                                  isem_s[q]).wait()
            pltpu.make_async_copy(dstp_hbm.at[pl.ds(0, SBB), :], didx[q],
                                  isem_d[q]).wait()

        def start_gather(slot, q, j):
            pltpu.async_copy(feat_hbm.at[sidx[q].at[jnp.int32(j)]],
                             rows[slot], gsem[slot])

        def wait_gather(slot, q, j):
            pltpu.make_async_copy(feat_hbm.at[sidx[q].at[jnp.int32(j)]],
                                  rows[slot], gsem[slot]).wait()

        start_idx(0, jnp.int32(0))
        start_idx(1, jnp.int32(1))
        wait_idx(0)
        for j in range(2):
            start_gather(j, 0, j)

        @pl.loop(jnp.int32(0), jnp.int32(nsb // 2))
        def _super(p):
            for q in range(2):
                sb = jnp.int32(2) * p + jnp.int32(q)
                for j in range(SBB):
                    slot = j % 2
                    wait_gather(slot, q, j)
                    pltpu.async_copy(rows[slot],
                                     h_s.at[didx[q].at[jnp.int32(j)]],
                                     ssem[slot], add=True)
                    ones = jnp.full((L,), 1, jnp.int32)
                    for u in range(B // L):
                        vd = didx[q][jnp.int32(j), pl.ds(jnp.int32(u * L), L)]
                        plsc.addupdate_scatter(hd_v, [vd], ones)
                    pltpu.make_async_copy(rows[slot],
                                          h_s.at[didx[q].at[jnp.int32(j)]],
                                          ssem[slot]).wait()
                    if j < SBB - 2:
                        start_gather(slot, q, j + 2)
                    else:
                        if j == SBB - 2:
                            wait_idx(1 - q)
                        start_gather(slot, 1 - q, j - (SBB - 2))
                start_idx(q, sb + jnp.int32(2))

        for j in range(2):
            wait_gather(j, nsb % 2, j)
        wait_idx((nsb - 1) % 2)

        plsc.subcore_barrier()

        r0 = s * jnp.int32(rows_pt)
        pltpu.async_copy(h_s.at[pl.ds(r0, rows_pt), :],
                         out_hbm.at[c, pl.ds(r0, rows_pt), :], gsem0).wait()
        pltpu.async_copy(hd_v, out_d_hbm.at[c, s], gsem1).wait()

    return agg_kernel


def kernel(x, edge_index):
    n, d = x.shape
    e = edge_index.shape[1]
    src = edge_index[0].astype(jnp.int32)
    dst = edge_index[1].astype(jnp.int32)

    n_pad = _round_up(n + 1, NS * B)
    e_pad = _round_up(e, 2 * NC * NS * B)
    pr = n_pad - n
    pad = e_pad - e
    pad_idx = n + (jnp.arange(pad, dtype=jnp.int32) % pr)
    extra = jnp.zeros((2048,), jnp.int32)
    srcp = jnp.concatenate([src, pad_idx, extra])
    dstp = jnp.concatenate([dst, pad_idx, extra])

    cnt_s = _degree_call(n_pad, e_pad)(srcp)
    deg_out = cnt_s.sum(axis=(0, 1))[:n].astype(jnp.float32) + 1.0

    feat = x * lax.rsqrt(deg_out)[:, None]
    featp = jnp.concatenate([feat, jnp.zeros((pr, d), jnp.float32)])

    srcp2 = srcp.reshape(-1, B)
    dstp2 = dstp.reshape(-1, B)
    hp, cnt_d = _agg_call(n_pad, e_pad, d)(featp, srcp2, dstp2)
    deg_in = cnt_d.sum(axis=(0, 1))[:n].astype(jnp.float32) + 1.0
    h = (hp[0, :n] + hp[1, :n]) * lax.rsqrt(deg_in)[:, None]
    return h
